# baseline (speedup 1.0000x reference)
import functools

import jax
import jax.numpy as jnp
from jax import lax
from jax.experimental import pallas as pl
from jax.experimental.pallas import tpu as pltpu
from jax.experimental.pallas import tpu_sc as plsc

BS_, L_, D_ = 16, 2048, 256
N_ = BS_ * L_
NW_ = 32
CHUNK_ = N_ // NW_
VB_ = 64
CP_ = 128
LANES_ = 16
DW_ = D_ // LANES_
TR_ = N_ // 8
HL_ = D_ // 128


FB_ = 512
NJ_ = L_ // FB_
JH_ = (CHUNK_ // FB_)


def _bits_body(x_ref, prev_ref, bits_ref, counts_ref):
    j = pl.program_id(1)
    xb = x_ref[0]
    pr = prev_ref[0, 7:8, :]
    rolled = pltpu.roll(xb, shift=1, axis=0)
    row_i = lax.broadcasted_iota(jnp.int32, (FB_, 1), 0)
    prev = jnp.where(row_i == 0, pr, rolled)
    ad = jnp.abs(xb - prev)
    s16 = jnp.sum(ad.reshape(FB_ // 128, 128, D_), axis=2)
    g_i = lax.broadcasted_iota(jnp.int32, (FB_ // 128, 128), 0)
    l_i = lax.broadcasted_iota(jnp.int32, (FB_ // 128, 128), 1)
    bit16 = jnp.where((s16 > 0.0) | ((j == 0) & (g_i == 0) & (l_i == 0)),
                      1, 0).astype(jnp.int32)
    bits_ref[0, pl.ds(j * (FB_ // 128), FB_ // 128), :] = \
        bit16.reshape(FB_ // 128, 128)
    part = jnp.sum(bit16)
    h_i = lax.broadcasted_iota(jnp.int32, (1, 2, 1), 1)
    part2 = jnp.where(h_i == j // JH_, part, 0)
    counts_ref[...] = jnp.where(j == 0, part2, counts_ref[...] + part2)


_bits_call = pl.pallas_call(
    _bits_body,
    grid=(BS_, NJ_),
    in_specs=[
        pl.BlockSpec((1, FB_, D_), lambda i, j: (i, j, 0)),
        pl.BlockSpec((1, 8, D_),
                     lambda i, j: (i, jnp.maximum(j * (FB_ // 8) - 1, 0), 0)),
    ],
    out_specs=[pl.BlockSpec((1, L_ // 128, 128), lambda i, j: (i, 0, 0)),
               pl.BlockSpec((1, 2, 1), lambda i, j: (i, 0, 0))],
    out_shape=(jax.ShapeDtypeStruct((BS_, L_ // 128, 128), jnp.int32),
               jax.ShapeDtypeStruct((BS_, 2, 1), jnp.int32)),
)


def _seg_body(x4, bits_hbm, cnts_hbm, out4, ids_hbm,
              bits_v, cnts_v, cp0, cpb, cpc, cp1, seg_buf, ids_v, cnt_s,
              ls0, ls1, ls2, ss0, ss1, ss2):
    i32 = jnp.int32
    wid = lax.axis_index("s") * 2 + lax.axis_index("c")
    cs = wid * CHUNK_
    half = wid & 1
    rs = (wid >> 1) * L_

    pltpu.sync_copy(bits_hbm.at[pl.ds(cs, CHUNK_)], bits_v.at[pl.ds(0, CHUNK_)])

    @pl.when(half == 0)
    def _load_next_bits():
        pltpu.sync_copy(bits_hbm.at[pl.ds(cs + CHUNK_, CHUNK_)],
                        bits_v.at[pl.ds(CHUNK_, CHUNK_)])

    pltpu.sync_copy(cnts_hbm, cnts_v)
    iota = lax.iota(i32, LANES_)
    v0 = cnts_v[pl.ds(0, LANES_)]
    v1 = cnts_v[pl.ds(LANES_, LANES_)]
    base = (jnp.sum(jnp.where(iota < wid, v0, 0))
            + jnp.sum(jnp.where(iota + LANES_ < wid, v1, 0)))
    mycnt = jnp.sum(jnp.where(iota == wid, v0, 0)) +         jnp.sum(jnp.where(iota + LANES_ == wid, v1, 0))
    nseg = jnp.sum(v0) + jnp.sum(v1)

    def ids_body(g, st):
        carry, lz = st
        bv = bits_v[pl.ds(g * LANES_, LANES_)]
        c = plsc.cumsum(bv) + (carry - (base - 1))
        ids_v[pl.ds(g * LANES_, LANES_)] = c + (base - 1)
        lz = lz + jnp.sum(jnp.where(c == 0, 1, 0).astype(i32))
        return carry + jnp.sum(bv), lz

    _, lz = lax.fori_loop(0, CHUNK_ // LANES_, ids_body,
                          (base - 1, jnp.zeros((), i32)))
    pltpu.sync_copy(ids_v, ids_hbm.at[pl.ds(cs, CHUNK_)])

    nxt = bits_v[pl.ds(CHUNK_, LANES_)][0]
    is_fast = ((mycnt == CHUNK_) & ((half == 1) | (nxt == 1))
               & ((base & 7) == 0))

    @pl.when(is_fast)
    def _fast():
        bufs = (cp0, cpb, cpc)
        lsem = (ls0, ls1, ls2)
        ssem = (ss0, ss1, ss2)
        nblk = CHUNK_ // CP_
        tpb = CP_ // 8

        def xblk(g):
            return x4.at[pl.ds((cs >> 3) + g * tpb, tpb)]

        def oblk(g):
            return out4.at[pl.ds((base >> 3) + g * tpb, tpb)]

        ld = {0: pltpu.async_copy(xblk(0), bufs[0], lsem[0]),
              1: pltpu.async_copy(xblk(1), bufs[1], lsem[1])}
        st = {}
        for g in range(nblk):
            bi = g % 3
            ld[g].wait()
            st[g] = pltpu.async_copy(bufs[bi], oblk(g), ssem[bi])
            if g >= 1:
                st[g - 1].wait()
            if g + 2 < nblk:
                nb = (g + 2) % 3
                ld[g + 2] = pltpu.async_copy(xblk(g + 2), bufs[nb], lsem[nb])
        st[nblk - 1].wait()

    def _emit_row(src_ref, src_word, out_row):
        for h in range(HL_):
            pltpu.sync_copy(src_ref.at[pl.ds(src_word + h * 128, 128)],
                            out4.at[out_row >> 3, h, out_row & 7])

    @pl.when(jnp.logical_not(is_fast) & (mycnt > 0))
    def _generic():
        p_start = cs + lz
        ne16 = jnp.where(half == 0, CHUNK_ // LANES_, 0)

        def lz2_body(g, st):
            c2, lz2 = st
            bv = bits_v[pl.ds(CHUNK_ + g * LANES_, LANES_)]
            c = plsc.cumsum(bv) + c2
            lz2 = lz2 + jnp.sum(jnp.where(c == 0, 1, 0).astype(i32))
            return c2 + jnp.sum(bv), lz2

        _, lz2 = lax.fori_loop(0, ne16, lz2_body,
                               (jnp.zeros((), i32), jnp.zeros((), i32)))
        p_end = cs + CHUNK_ + lz2

        def group(g16, st, fstart):
            bv = bits_v[pl.ds(fstart - cs + g16 * LANES_, LANES_)]

            def frame(jj, cur_slot):
                t = fstart + g16 * LANES_ + jj
                q = g16 * 2 + (jj >> 3)
                sl = jj & 7
                bit = bv[jj]
                inreg = (t >= p_start) & (t < p_end)

                def ld(k):
                    return cp0[q, k >> 3, sl, pl.ds((k & 7) * LANES_, LANES_)]

                def on_boundary():
                    ns = cur_slot + 1
                    for k in range(DW_):
                        seg_buf[pl.ds(ns * D_ + k * LANES_, LANES_)] = ld(k)
                    cnt_s[ns] = i32(1)
                    return ns

                def on_cont():
                    for k in range(DW_):
                        plsc.addupdate(
                            seg_buf.at[pl.ds(cur_slot * D_ + k * LANES_,
                                             LANES_)],
                            ld(k))
                    cnt_s[cur_slot] = cnt_s[cur_slot] + 1
                    return cur_slot

                return lax.cond(
                    inreg, lambda: lax.cond(bit == 1, on_boundary, on_cont),
                    lambda: cur_slot)

            cur_slot = st
            for jj in range(LANES_):
                cur_slot = frame(jj, cur_slot)
            return cur_slot

        def flush(n, out_base):
            def fr(r, _):
                inv = 1.0 / jnp.full((LANES_,), cnt_s[r].astype(jnp.float32))
                for k in range(DW_):
                    seg_buf[pl.ds(r * D_ + k * LANES_, LANES_)] = \
                        seg_buf[pl.ds(r * D_ + k * LANES_, LANES_)] * inv
                _emit_row(seg_buf, r * D_, out_base + r)
                return 0
            lax.fori_loop(0, n, fr, 0)

        def gen_block(g, st):
            cur_slot, flush_base = st
            fstart = rs + g * VB_
            pltpu.sync_copy(x4.at[pl.ds(fstart >> 3, VB_ // 8)],
                            cp0.at[pl.ds(0, VB_ // 8)])
            cur_slot = lax.fori_loop(
                0, VB_ // LANES_, lambda g16, s: group(g16, s, fstart),
                cur_slot)

            def do_flush():
                flush(cur_slot, flush_base)
                for k in range(DW_):
                    seg_buf[pl.ds(k * LANES_, LANES_)] = \
                        seg_buf[pl.ds(cur_slot * D_ + k * LANES_, LANES_)]
                cnt_s[0] = cnt_s[cur_slot]
                return i32(0), flush_base + cur_slot

            return lax.cond(cur_slot >= 0, do_flush,
                            lambda: (cur_slot, flush_base))

        g0 = (p_start - rs) >> 6
        g1 = (p_end - rs + VB_ - 1) >> 6
        cur_slot, flush_base = lax.fori_loop(
            g0, g1, gen_block, (jnp.full((), -1, i32), base))

        @pl.when(cur_slot >= 0)
        def _final():
            flush(cur_slot + 1, flush_base)

    tail = N_ - nseg

    @pl.when(tail > 0)
    def _zero():
        share = (tail + NW_ - 1) >> 5
        zs = nseg + wid * share
        nrows = jnp.maximum(jnp.minimum(zs + share, N_) - zs, 0)
        for k in range(DW_):
            cp1[pl.ds(k * LANES_, LANES_)] = jnp.zeros((LANES_,), jnp.float32)

        def zb(r, _):
            _emit_row(cp1, 0, zs + r)
            return 0
        lax.fori_loop(0, nrows, zb, 0)


@functools.cache
def _make_seg_call():
    return pl.kernel(
        _seg_body,
        out_type=(jax.ShapeDtypeStruct((TR_, HL_, 8, 128), jnp.float32),
                  jax.ShapeDtypeStruct((N_,), jnp.int32)),
        mesh=plsc.VectorSubcoreMesh(core_axis_name="c", subcore_axis_name="s"),
        compiler_params=pltpu.CompilerParams(needs_layout_passes=False,
                                             use_tc_tiling_on_sc=False),
        scratch_types=[
            pltpu.VMEM((2 * CHUNK_,), jnp.int32),
            pltpu.VMEM((NW_,), jnp.int32),
            pltpu.VMEM((CP_ // 8, HL_, 8, 128), jnp.float32),
            pltpu.VMEM((CP_ // 8, HL_, 8, 128), jnp.float32),
            pltpu.VMEM((CP_ // 8, HL_, 8, 128), jnp.float32),
            pltpu.VMEM((D_,), jnp.float32),
            pltpu.VMEM(((VB_ + 1) * D_,), jnp.float32),
            pltpu.VMEM((CHUNK_,), jnp.int32),
            pltpu.SMEM((VB_ + 1,), jnp.int32),
            pltpu.SemaphoreType.DMA,
            pltpu.SemaphoreType.DMA,
            pltpu.SemaphoreType.DMA,
            pltpu.SemaphoreType.DMA,
            pltpu.SemaphoreType.DMA,
            pltpu.SemaphoreType.DMA,
        ],
    )


def kernel(cFeatures, encodedData, label):
    x = encodedData
    bits3, counts3 = _bits_call(x, x)
    bits = bits3.reshape(N_)
    counts = counts3.reshape(NW_)
    x4 = x.reshape(TR_, 8, HL_, 128).transpose(0, 2, 1, 3)
    out4, seg_ids = _make_seg_call()(x4, bits, counts)
    seg_avg = out4.transpose(0, 2, 1, 3).reshape(N_, D_)
    return seg_avg, seg_ids


# speedup vs baseline: 1.4031x; 1.4031x over previous
import functools

import jax
import jax.numpy as jnp
from jax import lax
from jax.experimental import pallas as pl
from jax.experimental.pallas import tpu as pltpu
from jax.experimental.pallas import tpu_sc as plsc

BS_, L_, D_ = 16, 2048, 256
N_ = BS_ * L_
NW_ = 32
CHUNK_ = N_ // NW_
VB_ = 64
CP_ = 128
LANES_ = 16
DW_ = D_ // LANES_
TR_ = N_ // 8
HL_ = D_ // 128


def _bits_body(x_ref, bits_ref, counts_ref):
    xb = x_ref[0]
    prev = pltpu.roll(xb, shift=1, axis=0)
    ad = jnp.abs(xb - prev)
    s16 = jnp.sum(ad.reshape(L_ // 128, 128, D_), axis=2)
    g_i = lax.broadcasted_iota(jnp.int32, (L_ // 128, 128), 0)
    l_i = lax.broadcasted_iota(jnp.int32, (L_ // 128, 128), 1)
    bit16 = jnp.where((s16 > 0.0) | ((g_i == 0) & (l_i == 0)),
                      1, 0).astype(jnp.int32)
    bits_ref[...] = bit16.reshape(1, L_ // 128, 128)
    c128 = jnp.sum(bit16, axis=1, keepdims=True)
    c2 = jnp.sum(c128.reshape(2, 8, 1), axis=1)
    counts_ref[...] = c2.reshape(1, 2, 1)


_bits_call = pl.pallas_call(
    _bits_body,
    grid=(BS_,),
    in_specs=[pl.BlockSpec((1, L_, D_), lambda i: (i, 0, 0))],
    out_specs=[pl.BlockSpec((1, L_ // 128, 128), lambda i: (i, 0, 0)),
               pl.BlockSpec((1, 2, 1), lambda i: (i, 0, 0))],
    out_shape=(jax.ShapeDtypeStruct((BS_, L_ // 128, 128), jnp.int32),
               jax.ShapeDtypeStruct((BS_, 2, 1), jnp.int32)),
)


def _seg_body(x4, bits_hbm, cnts_hbm, out4, ids_hbm,
              bits_v, cnts_v, cp0, cpb, cpc, cp1, seg_buf, ids_v, cnt_s,
              ls0, ls1, ls2, ss0, ss1, ss2):
    i32 = jnp.int32
    wid = lax.axis_index("s") * 2 + lax.axis_index("c")
    cs = wid * CHUNK_
    half = wid & 1
    rs = (wid >> 1) * L_

    pltpu.sync_copy(bits_hbm.at[pl.ds(cs, CHUNK_)], bits_v.at[pl.ds(0, CHUNK_)])

    @pl.when(half == 0)
    def _load_next_bits():
        pltpu.sync_copy(bits_hbm.at[pl.ds(cs + CHUNK_, CHUNK_)],
                        bits_v.at[pl.ds(CHUNK_, CHUNK_)])

    pltpu.sync_copy(cnts_hbm, cnts_v)
    iota = lax.iota(i32, LANES_)
    v0 = cnts_v[pl.ds(0, LANES_)]
    v1 = cnts_v[pl.ds(LANES_, LANES_)]
    base = (jnp.sum(jnp.where(iota < wid, v0, 0))
            + jnp.sum(jnp.where(iota + LANES_ < wid, v1, 0)))
    mycnt = jnp.sum(jnp.where(iota == wid, v0, 0)) +         jnp.sum(jnp.where(iota + LANES_ == wid, v1, 0))
    nseg = jnp.sum(v0) + jnp.sum(v1)

    def ids_body(g, st):
        carry, lz = st
        bv = bits_v[pl.ds(g * LANES_, LANES_)]
        c = plsc.cumsum(bv) + (carry - (base - 1))
        ids_v[pl.ds(g * LANES_, LANES_)] = c + (base - 1)
        lz = lz + jnp.sum(jnp.where(c == 0, 1, 0).astype(i32))
        return carry + jnp.sum(bv), lz

    _, lz = lax.fori_loop(0, CHUNK_ // LANES_, ids_body,
                          (base - 1, jnp.zeros((), i32)))
    pltpu.sync_copy(ids_v, ids_hbm.at[pl.ds(cs, CHUNK_)])

    nxt = bits_v[pl.ds(CHUNK_, LANES_)][0]
    is_fast = ((mycnt == CHUNK_) & ((half == 1) | (nxt == 1))
               & ((base & 7) == 0))

    @pl.when(is_fast)
    def _fast():
        bufs = (cp0, cpb, cpc)
        lsem = (ls0, ls1, ls2)
        ssem = (ss0, ss1, ss2)
        nblk = CHUNK_ // CP_
        tpb = CP_ // 8

        def xblk(g):
            return x4.at[pl.ds((cs >> 3) + g * tpb, tpb)]

        def oblk(g):
            return out4.at[pl.ds((base >> 3) + g * tpb, tpb)]

        ld = {0: pltpu.async_copy(xblk(0), bufs[0], lsem[0]),
              1: pltpu.async_copy(xblk(1), bufs[1], lsem[1])}
        st = {}
        for g in range(nblk):
            bi = g % 3
            ld[g].wait()
            st[g] = pltpu.async_copy(bufs[bi], oblk(g), ssem[bi])
            if g >= 1:
                st[g - 1].wait()
            if g + 2 < nblk:
                nb = (g + 2) % 3
                ld[g + 2] = pltpu.async_copy(xblk(g + 2), bufs[nb], lsem[nb])
        st[nblk - 1].wait()

    def _emit_row(src_ref, src_word, out_row):
        for h in range(HL_):
            pltpu.sync_copy(src_ref.at[pl.ds(src_word + h * 128, 128)],
                            out4.at[out_row >> 3, h, out_row & 7])

    @pl.when(jnp.logical_not(is_fast) & (mycnt > 0))
    def _generic():
        p_start = cs + lz
        ne16 = jnp.where(half == 0, CHUNK_ // LANES_, 0)

        def lz2_body(g, st):
            c2, lz2 = st
            bv = bits_v[pl.ds(CHUNK_ + g * LANES_, LANES_)]
            c = plsc.cumsum(bv) + c2
            lz2 = lz2 + jnp.sum(jnp.where(c == 0, 1, 0).astype(i32))
            return c2 + jnp.sum(bv), lz2

        _, lz2 = lax.fori_loop(0, ne16, lz2_body,
                               (jnp.zeros((), i32), jnp.zeros((), i32)))
        p_end = cs + CHUNK_ + lz2

        def group(g16, st, fstart):
            bv = bits_v[pl.ds(fstart - cs + g16 * LANES_, LANES_)]

            def frame(jj, cur_slot):
                t = fstart + g16 * LANES_ + jj
                q = g16 * 2 + (jj >> 3)
                sl = jj & 7
                bit = bv[jj]
                inreg = (t >= p_start) & (t < p_end)

                def ld(k):
                    return cp0[q, k >> 3, sl, pl.ds((k & 7) * LANES_, LANES_)]

                def on_boundary():
                    ns = cur_slot + 1
                    for k in range(DW_):
                        seg_buf[pl.ds(ns * D_ + k * LANES_, LANES_)] = ld(k)
                    cnt_s[ns] = i32(1)
                    return ns

                def on_cont():
                    for k in range(DW_):
                        plsc.addupdate(
                            seg_buf.at[pl.ds(cur_slot * D_ + k * LANES_,
                                             LANES_)],
                            ld(k))
                    cnt_s[cur_slot] = cnt_s[cur_slot] + 1
                    return cur_slot

                return lax.cond(
                    inreg, lambda: lax.cond(bit == 1, on_boundary, on_cont),
                    lambda: cur_slot)

            cur_slot = st
            for jj in range(LANES_):
                cur_slot = frame(jj, cur_slot)
            return cur_slot

        def flush(n, out_base):
            def fr(r, _):
                inv = 1.0 / jnp.full((LANES_,), cnt_s[r].astype(jnp.float32))
                for k in range(DW_):
                    seg_buf[pl.ds(r * D_ + k * LANES_, LANES_)] = \
                        seg_buf[pl.ds(r * D_ + k * LANES_, LANES_)] * inv
                _emit_row(seg_buf, r * D_, out_base + r)
                return 0
            lax.fori_loop(0, n, fr, 0)

        def gen_block(g, st):
            cur_slot, flush_base = st
            fstart = rs + g * VB_
            pltpu.sync_copy(x4.at[pl.ds(fstart >> 3, VB_ // 8)],
                            cp0.at[pl.ds(0, VB_ // 8)])
            cur_slot = lax.fori_loop(
                0, VB_ // LANES_, lambda g16, s: group(g16, s, fstart),
                cur_slot)

            def do_flush():
                flush(cur_slot, flush_base)
                for k in range(DW_):
                    seg_buf[pl.ds(k * LANES_, LANES_)] = \
                        seg_buf[pl.ds(cur_slot * D_ + k * LANES_, LANES_)]
                cnt_s[0] = cnt_s[cur_slot]
                return i32(0), flush_base + cur_slot

            return lax.cond(cur_slot >= 0, do_flush,
                            lambda: (cur_slot, flush_base))

        g0 = (p_start - rs) >> 6
        g1 = (p_end - rs + VB_ - 1) >> 6
        cur_slot, flush_base = lax.fori_loop(
            g0, g1, gen_block, (jnp.full((), -1, i32), base))

        @pl.when(cur_slot >= 0)
        def _final():
            flush(cur_slot + 1, flush_base)

    tail = N_ - nseg

    @pl.when(tail > 0)
    def _zero():
        share = (tail + NW_ - 1) >> 5
        zs = nseg + wid * share
        nrows = jnp.maximum(jnp.minimum(zs + share, N_) - zs, 0)
        for k in range(DW_):
            cp1[pl.ds(k * LANES_, LANES_)] = jnp.zeros((LANES_,), jnp.float32)

        def zb(r, _):
            _emit_row(cp1, 0, zs + r)
            return 0
        lax.fori_loop(0, nrows, zb, 0)


@functools.cache
def _make_seg_call():
    return pl.kernel(
        _seg_body,
        out_type=(jax.ShapeDtypeStruct((TR_, HL_, 8, 128), jnp.float32),
                  jax.ShapeDtypeStruct((N_,), jnp.int32)),
        mesh=plsc.VectorSubcoreMesh(core_axis_name="c", subcore_axis_name="s"),
        compiler_params=pltpu.CompilerParams(needs_layout_passes=False,
                                             use_tc_tiling_on_sc=False),
        scratch_types=[
            pltpu.VMEM((2 * CHUNK_,), jnp.int32),
            pltpu.VMEM((NW_,), jnp.int32),
            pltpu.VMEM((CP_ // 8, HL_, 8, 128), jnp.float32),
            pltpu.VMEM((CP_ // 8, HL_, 8, 128), jnp.float32),
            pltpu.VMEM((CP_ // 8, HL_, 8, 128), jnp.float32),
            pltpu.VMEM((D_,), jnp.float32),
            pltpu.VMEM(((VB_ + 1) * D_,), jnp.float32),
            pltpu.VMEM((CHUNK_,), jnp.int32),
            pltpu.SMEM((VB_ + 1,), jnp.int32),
            pltpu.SemaphoreType.DMA,
            pltpu.SemaphoreType.DMA,
            pltpu.SemaphoreType.DMA,
            pltpu.SemaphoreType.DMA,
            pltpu.SemaphoreType.DMA,
            pltpu.SemaphoreType.DMA,
        ],
    )


def kernel(cFeatures, encodedData, label):
    x = encodedData
    bits3, counts3 = _bits_call(x)
    bits = bits3.reshape(N_)
    counts = counts3.reshape(NW_)
    x4 = x.reshape(TR_, 8, HL_, 128).transpose(0, 2, 1, 3)
    out4, seg_ids = _make_seg_call()(x4, bits, counts)
    seg_avg = out4.transpose(0, 2, 1, 3).reshape(N_, D_)
    return seg_avg, seg_ids
